# Initial kernel scaffold; baseline (speedup 1.0000x reference)
import jax
import jax.numpy as jnp
from jax.experimental import pallas as pl


def kernel(charge, spin, task, charge_table, spin_table, task_table, bias):
    raise NotImplementedError("write your pallas kernel here")


# speedup vs baseline: 1.9744x; 1.9744x over previous
import functools

import jax
import jax.numpy as jnp
from jax import lax
from jax.experimental import pallas as pl
from jax.experimental.pallas import tpu as pltpu
from jax.experimental.pallas import tpu_sc as plsc

B = 16384
C = 128
NC = 2
NS = 16
NW = NC * NS
ROWS_W = B // NW
CHUNK = 128
NCHUNK = ROWS_W // CHUNK
LANES = 16

_mesh = plsc.VectorSubcoreMesh(core_axis_name="c", subcore_axis_name="s")


@functools.partial(
    pl.kernel,
    out_type=jax.ShapeDtypeStruct((B, C), jnp.float32),
    mesh=_mesh,
    scratch_types=[
        pltpu.VMEM((NCHUNK, CHUNK), jnp.int32),
        pltpu.VMEM((NCHUNK, CHUNK), jnp.int32),
        pltpu.VMEM((NCHUNK, CHUNK), jnp.int32),
        pltpu.VMEM((CHUNK, C), jnp.float32),
        pltpu.VMEM((CHUNK, C), jnp.float32),
        pltpu.VMEM((CHUNK, C), jnp.float32),
        pltpu.SemaphoreType.DMA,
    ],
)
def _embed_silu(cidx_hbm, sidx_hbm, tidx_hbm, ctab_hbm, stab_hbm, ttab_hbm,
                out_hbm, cidx_v, sidx_v, tidx_v, abuf, bbuf, cbuf, sem):
    wid = lax.axis_index("s") * NC + lax.axis_index("c")
    pltpu.sync_copy(cidx_hbm.at[wid], cidx_v)
    pltpu.sync_copy(sidx_hbm.at[wid], sidx_v)
    pltpu.sync_copy(tidx_hbm.at[wid], tidx_v)

    for j in range(NCHUNK):
        cdma = pltpu.async_copy(ctab_hbm.at[cidx_v.at[j]], abuf, sem)
        sdma = pltpu.async_copy(stab_hbm.at[sidx_v.at[j]], bbuf, sem)
        tdma = pltpu.async_copy(ttab_hbm.at[tidx_v.at[j]], cbuf, sem)
        cdma.wait()
        sdma.wait()
        tdma.wait()

        def row_body(i, carry):
            for c8 in range(C // LANES):
                sl = pl.ds(c8 * LANES, LANES)
                x = abuf[i, sl] + bbuf[i, sl] + cbuf[i, sl]
                abuf[i, sl] = x / (1.0 + jnp.exp(-x))
            return carry

        lax.fori_loop(0, CHUNK, row_body, 0)
        pltpu.sync_copy(abuf, out_hbm.at[pl.ds(wid * ROWS_W + j * CHUNK, CHUNK)])


def kernel(charge, spin, task, charge_table, spin_table, task_table, bias):
    cidx = (charge + 100).reshape(NW, NCHUNK, CHUNK)
    sidx = spin.reshape(NW, NCHUNK, CHUNK)
    tidx = task.reshape(NW, NCHUNK, CHUNK)
    ttab = task_table + bias[None, :]
    return _embed_silu(cidx, sidx, tidx, charge_table, spin_table, ttab)


# speedup vs baseline: 2.0444x; 1.0355x over previous
import functools

import jax
import jax.numpy as jnp
from jax import lax
from jax.experimental import pallas as pl
from jax.experimental.pallas import tpu as pltpu
from jax.experimental.pallas import tpu_sc as plsc

B = 16384
C = 128
NC = 2
NS = 16
NW = NC * NS
ROWS_W = B // NW
CHUNK = 128
NCHUNK = ROWS_W // CHUNK
LANES = 16

_mesh = plsc.VectorSubcoreMesh(core_axis_name="c", subcore_axis_name="s")


@functools.partial(
    pl.kernel,
    out_type=jax.ShapeDtypeStruct((B, C), jnp.float32),
    mesh=_mesh,
    scratch_types=[
        pltpu.VMEM((NCHUNK, CHUNK), jnp.int32),
        pltpu.VMEM((NCHUNK, CHUNK), jnp.int32),
        pltpu.VMEM((NCHUNK, CHUNK), jnp.int32),
        [pltpu.VMEM((CHUNK, C), jnp.float32) for _ in range(2)],
        [pltpu.VMEM((CHUNK, C), jnp.float32) for _ in range(2)],
        [pltpu.VMEM((CHUNK, C), jnp.float32) for _ in range(2)],
        [pltpu.SemaphoreType.DMA for _ in range(2)],
        pltpu.SemaphoreType.DMA,
    ],
)
def _embed_silu(cidx_hbm, sidx_hbm, tidx_hbm, ctab_hbm, stab_hbm, ttab_hbm,
                out_hbm, cidx_v, sidx_v, tidx_v, abuf, bbuf, cbuf, gsem, osem):
    wid = lax.axis_index("s") * NC + lax.axis_index("c")
    pltpu.sync_copy(cidx_hbm.at[wid], cidx_v)
    pltpu.sync_copy(sidx_hbm.at[wid], sidx_v)
    pltpu.sync_copy(tidx_hbm.at[wid], tidx_v)

    def start_gather(j):
        s = j % 2
        return (
            pltpu.async_copy(ctab_hbm.at[cidx_v.at[j]], abuf[s], gsem[s]),
            pltpu.async_copy(stab_hbm.at[sidx_v.at[j]], bbuf[s], gsem[s]),
            pltpu.async_copy(ttab_hbm.at[tidx_v.at[j]], cbuf[s], gsem[s]),
        )

    gathers = {0: start_gather(0)}
    outs = {}
    for j in range(NCHUNK):
        s = j % 2
        if j + 1 < NCHUNK:
            if j - 1 in outs:
                outs.pop(j - 1).wait()
            gathers[j + 1] = start_gather(j + 1)
        for d in gathers.pop(j):
            d.wait()

        def row_body(i, carry):
            for c8 in range(C // LANES):
                sl = pl.ds(c8 * LANES, LANES)
                x = abuf[s][i, sl] + bbuf[s][i, sl] + cbuf[s][i, sl]
                abuf[s][i, sl] = x / (1.0 + jnp.exp(-x))
            return carry

        lax.fori_loop(0, CHUNK, row_body, 0)
        outs[j] = pltpu.async_copy(
            abuf[s], out_hbm.at[pl.ds(wid * ROWS_W + j * CHUNK, CHUNK)], osem)
    for j in sorted(outs):
        outs.pop(j).wait()


def kernel(charge, spin, task, charge_table, spin_table, task_table, bias):
    cidx = (charge + 100).reshape(NW, NCHUNK, CHUNK)
    sidx = spin.reshape(NW, NCHUNK, CHUNK)
    tidx = task.reshape(NW, NCHUNK, CHUNK)
    ttab = task_table + bias[None, :]
    return _embed_silu(cidx, sidx, tidx, charge_table, spin_table, ttab)
